# baseline (speedup 1.0000x reference)
import functools

import jax
import jax.numpy as jnp
from jax import lax
from jax.experimental import pallas as pl
from jax.experimental.pallas import tpu as pltpu
from jax.experimental.pallas import tpu_sc as plsc

B = 16384
D = 32
NC = 2
NS = 16
NW = NC * NS
BPW = B // NW
CHUNK = 128
NCHUNK = BPW // CHUNK
GROUPS = BPW // 16


def _gmf_body(users_hbm, items_hbm, ut_hbm, it_hbm, w_hbm, b_hbm, out_hbm,
              uidx_v, iidx_v, urows_v, irows_v, w_v, b_v, out_v,
              sem_u, sem_i):
    wid = lax.axis_index("s") * NC + lax.axis_index("c")
    base = wid * BPW

    pltpu.sync_copy(users_hbm.at[pl.ds(base, BPW)], uidx_v)
    pltpu.sync_copy(items_hbm.at[pl.ds(base, BPW)], iidx_v)
    pltpu.sync_copy(w_hbm, w_v)
    pltpu.sync_copy(b_hbm, b_v.at[pl.ds(0, 1)])

    copies = []
    for c in range(NCHUNK):
        sl = pl.ds(c * CHUNK, CHUNK)
        copies.append(pltpu.async_copy(
            ut_hbm.at[uidx_v.at[sl]], urows_v.at[sl], sem_u))
        copies.append(pltpu.async_copy(
            it_hbm.at[iidx_v.at[sl]], irows_v.at[sl], sem_i))
    for cp in copies:
        cp.wait()

    iota16 = lax.iota(jnp.int32, 16)
    wv0 = w_v[0, pl.ds(0, 16)]
    wv1 = w_v[0, pl.ds(16, 16)]
    bias = b_v[pl.ds(0, 16)][0]

    def group(g, carry):
        rows = g * 16 + iota16
        acc = jnp.zeros((16,), jnp.float32)
        for d in range(D):
            dvec = jnp.full((16,), d, jnp.int32)
            gu = plsc.load_gather(urows_v, [rows, dvec])
            gi = plsc.load_gather(irows_v, [rows, dvec])
            wd = wv0[d] if d < 16 else wv1[d - 16]
            acc = acc + (gu * gi) * wd
        out_v[pl.ds(g * 16, 16)] = acc + bias
        return carry

    lax.fori_loop(0, GROUPS, group, 0)

    pltpu.sync_copy(out_v, out_hbm.at[pl.ds(base, BPW)])


@jax.jit
def _gmf(users, items, user_table, item_table, W, b):
    mesh = plsc.VectorSubcoreMesh(core_axis_name="c", subcore_axis_name="s")
    f = pl.kernel(
        _gmf_body,
        out_type=jax.ShapeDtypeStruct((B,), jnp.float32),
        mesh=mesh,
        scratch_types=[
            pltpu.VMEM((BPW,), jnp.int32),
            pltpu.VMEM((BPW,), jnp.int32),
            pltpu.VMEM((BPW, D), jnp.float32),
            pltpu.VMEM((BPW, D), jnp.float32),
            pltpu.VMEM((1, D), jnp.float32),
            pltpu.VMEM((16,), jnp.float32),
            pltpu.VMEM((BPW,), jnp.float32),
            pltpu.SemaphoreType.DMA,
            pltpu.SemaphoreType.DMA,
        ],
        compiler_params=pltpu.CompilerParams(
            needs_layout_passes=False, use_tc_tiling_on_sc=False),
    )
    return f(users, items, user_table, item_table, W, b)


def kernel(users, items, user_table, item_table, W, b):
    return _gmf(users, items, user_table, item_table, W, b)


# speedup vs baseline: 3.7308x; 3.7308x over previous
import functools

import jax
import jax.numpy as jnp
from jax import lax
from jax.experimental import pallas as pl
from jax.experimental.pallas import tpu as pltpu
from jax.experimental.pallas import tpu_sc as plsc

B = 16384
D = 32
NC = 2
NS = 16
NW = NC * NS
BPW = B // NW
GROUPS = BPW // 16
CHUNK = 8


def _gmf_body(users_hbm, items_hbm, ut_hbm, it_hbm, w_hbm, b_hbm, out_hbm,
              uidx_v, iidx_v, win_v, uc_v, w_v, b_v, out_v,
              sem_u, sem_i):
    wid = lax.axis_index("s") * NC + lax.axis_index("c")
    base = wid * BPW

    pltpu.sync_copy(users_hbm.at[pl.ds(base, BPW)], uidx_v)
    pltpu.sync_copy(items_hbm.at[pl.ds(base, BPW)], iidx_v)
    pltpu.sync_copy(w_hbm, w_v)
    pltpu.sync_copy(b_hbm, b_v.at[pl.ds(0, 1)])

    wv0 = w_v[0, pl.ds(0, 16)]
    wv1 = w_v[0, pl.ds(16, 16)]
    bias = b_v[pl.ds(0, 16)][0]
    lane16 = lax.iota(jnp.int32, 16)

    def fire(tab_hbm, ivec, sem):
        copies = []
        for j in range(16):
            c = ivec[j]
            c128 = pl.multiple_of((c >> 7) << 7, 128)
            copies.append(pltpu.async_copy(
                tab_hbm.at[:, pl.ds(c128, 128)], win_v.at[j], sem))
        return copies

    def group(g, carry):
        iv_u = uidx_v[pl.ds(g * 16, 16)]
        iv_i = iidx_v[pl.ds(g * 16, 16)]
        lv_u = iv_u & 127
        lv_i = iv_i & 127

        for cp in fire(ut_hbm, iv_u, sem_u):
            cp.wait()
        for d in range(D):
            dsplat = jnp.full((16,), d, jnp.int32)
            uc_v[d] = plsc.load_gather(win_v, [lane16, dsplat, lv_u])

        for cp in fire(it_hbm, iv_i, sem_i):
            cp.wait()
        acc = jnp.full((16,), bias, jnp.float32)
        for d in range(D):
            dsplat = jnp.full((16,), d, jnp.int32)
            i_d = plsc.load_gather(win_v, [lane16, dsplat, lv_i])
            wd = wv0[d] if d < 16 else wv1[d - 16]
            acc = acc + (uc_v[d] * i_d) * wd
        out_v[pl.ds(g * 16, 16)] = acc
        return carry

    lax.fori_loop(0, GROUPS, group, 0)

    pltpu.sync_copy(out_v, out_hbm.at[pl.ds(base, BPW)])


@jax.jit
def _gmf(users, items, user_table, item_table, W, b):
    mesh = plsc.VectorSubcoreMesh(core_axis_name="c", subcore_axis_name="s")
    f = pl.kernel(
        _gmf_body,
        out_type=jax.ShapeDtypeStruct((B,), jnp.float32),
        mesh=mesh,
        scratch_types=[
            pltpu.VMEM((BPW,), jnp.int32),
            pltpu.VMEM((BPW,), jnp.int32),
            pltpu.VMEM((16, D, 128), jnp.float32),
            pltpu.VMEM((D, 16), jnp.float32),
            pltpu.VMEM((1, D), jnp.float32),
            pltpu.VMEM((16,), jnp.float32),
            pltpu.VMEM((BPW,), jnp.float32),
            pltpu.SemaphoreType.DMA,
            pltpu.SemaphoreType.DMA,
        ],
        compiler_params=pltpu.CompilerParams(needs_layout_passes=False),
    )
    return f(users, items, user_table.T, item_table.T, W, b)


def kernel(users, items, user_table, item_table, W, b):
    return _gmf(users, items, user_table, item_table, W, b)
